# baseline (speedup 1.0000x reference)
import functools

import jax
import jax.numpy as jnp
from jax import lax
from jax.experimental import pallas as pl
from jax.experimental.pallas import tpu as pltpu
from jax.experimental.pallas import tpu_sc as plsc

NP = 10240
D = 256
H = 128
G = 64
BN = 1024
NB = NP // BN
NSUB = 16
CHUNK = 128
CPT = 80
IB = 16
NBLK = CPT // IB
NBUF = 2
EP = NSUB * CPT * CHUNK
ROWS_PER_TILE = NP // NSUB
NACC = NP + 8

_MAXNORM = 1.0 - 4e-3
_EPS = 1e-15


def _rnorm(x):
    return jnp.maximum(jnp.sqrt(jnp.sum(x * x, axis=-1, keepdims=True)), _EPS)


def _artanh(x):
    x = jnp.clip(x, -1.0 + 1e-7, 1.0 - 1e-7)
    return 0.5 * jnp.log((1.0 + x) / (1.0 - x))


def _proj(x):
    n = _rnorm(x)
    return x * jnp.minimum(1.0, _MAXNORM / n)


def _proj_expmap0(u):
    un = _rnorm(u)
    return u * (jnp.minimum(jnp.tanh(un), _MAXNORM) / un)


def _logmap0(p):
    pn = _rnorm(p)
    return p * (_artanh(pn) / pn)


def _proj_mobius_add(x, y):
    x2 = jnp.sum(x * x, -1, keepdims=True)
    y2 = jnp.sum(y * y, -1, keepdims=True)
    xy = jnp.sum(x * y, -1, keepdims=True)
    num = (1.0 + 2.0 * xy + y2) * x + (1.0 - x2) * y
    den = jnp.maximum(1.0 + 2.0 * xy + x2 * y2, _EPS)
    nn = _rnorm(num)
    return num * jnp.minimum(1.0 / den, _MAXNORM / nn)


def _proj_mobius_matvec(w, x):
    xn = _rnorm(x)
    mx = lax.dot_general(x, w, (((1,), (1,)), ((), ())),
                         preferred_element_type=jnp.float32)
    mxn = _rnorm(mx)
    return mx * (jnp.minimum(jnp.tanh(mxn / xn * _artanh(xn)), _MAXNORM) / mxn)


def _hyp_linear(w, b, h):
    mv = _proj_mobius_matvec(w, h)
    hb = _proj_expmap0(b)
    return _proj_mobius_add(mv, hb)


def _post_agg(agg):
    h = _proj_expmap0(agg)
    ht = jax.nn.relu(_logmap0(h))
    return _proj_expmap0(ht)


def _tc_in_body(x_ref, w_ref, b_ref, o0_ref, o1_ref):
    h = _proj_expmap0(x_ref[...])
    h = _hyp_linear(w_ref[...], b_ref[...], h)
    ht = _logmap0(h)
    o0_ref[...] = ht[:, :H]
    o1_ref[...] = ht[:, H:]


def _tc_in(xp, w, b):
    return pl.pallas_call(
        _tc_in_body,
        grid=(NB,),
        in_specs=[
            pl.BlockSpec((BN, D), lambda i: (i, 0)),
            pl.BlockSpec((D, D), lambda i: (0, 0)),
            pl.BlockSpec((1, D), lambda i: (0, 0)),
        ],
        out_specs=[
            pl.BlockSpec((BN, H), lambda i: (i, 0)),
            pl.BlockSpec((BN, H), lambda i: (i, 0)),
        ],
        out_shape=[jax.ShapeDtypeStruct((NP, H), jnp.float32)] * 2,
    )(xp, w, b)


def _tc_mid_body(a0_ref, a1_ref, w_ref, b_ref, o0_ref, o1_ref):
    agg = jnp.concatenate([a0_ref[...], a1_ref[...]], axis=1)
    h = _post_agg(agg)
    h = _hyp_linear(w_ref[...], b_ref[...], h)
    ht = _logmap0(h)
    o0_ref[...] = ht[:, :H]
    o1_ref[...] = ht[:, H:]


def _tc_mid(a0, a1, w, b):
    return pl.pallas_call(
        _tc_mid_body,
        grid=(NB,),
        in_specs=[
            pl.BlockSpec((BN, H), lambda i: (i, 0)),
            pl.BlockSpec((BN, H), lambda i: (i, 0)),
            pl.BlockSpec((D, D), lambda i: (0, 0)),
            pl.BlockSpec((1, D), lambda i: (0, 0)),
        ],
        out_specs=[
            pl.BlockSpec((BN, H), lambda i: (i, 0)),
            pl.BlockSpec((BN, H), lambda i: (i, 0)),
        ],
        out_shape=[jax.ShapeDtypeStruct((NP, H), jnp.float32)] * 2,
    )(a0, a1, w, b)


def _tc_pool_body(a0_ref, a1_ref, batch_ref, gw_ref, out_ref,
                  smax_s, den_s, num_s):
    p = pl.program_id(0)
    j = pl.program_id(1)

    agg = jnp.concatenate([a0_ref[...], a1_ref[...]], axis=1)
    h = _post_agg(agg)
    gw = gw_ref[...]
    gl = lax.dot_general(gw, h, (((1,), (1,)), ((), ())),
                         preferred_element_type=jnp.float32)
    b2d = batch_ref[...].reshape(1, BN)
    seg = lax.broadcasted_iota(jnp.int32, (G, BN), 0)
    mask = seg == b2d

    @pl.when(jnp.logical_and(p == 0, j == 0))
    def _():
        smax_s[...] = jnp.full_like(smax_s[...], -1e30)

    @pl.when(p == 0)
    def _():
        bm = jnp.max(jnp.where(mask, gl, -1e30), axis=1, keepdims=True)
        smax_s[...] = jnp.maximum(smax_s[...], bm)

    @pl.when(jnp.logical_and(p == 1, j == 0))
    def _():
        den_s[...] = jnp.zeros_like(den_s[...])
        num_s[...] = jnp.zeros_like(num_s[...])

    @pl.when(p == 1)
    def _():
        m = jnp.max(smax_s[...], axis=1, keepdims=True)
        e = jnp.where(mask, jnp.exp(gl - m), 0.0)
        den_s[...] += jnp.sum(e, axis=1, keepdims=True)
        num_s[...] += lax.dot_general(e, h, (((1,), (0,)), ((), ())),
                                      preferred_element_type=jnp.float32)

    @pl.when(jnp.logical_and(p == 1, j == NB - 1))
    def _():
        den = jnp.max(den_s[...], axis=1, keepdims=True)
        out_ref[...] = num_s[...] / (den + 1e-16)


def _tc_pool(a0, a1, batch3, gw):
    return pl.pallas_call(
        _tc_pool_body,
        grid=(2, NB),
        in_specs=[
            pl.BlockSpec((BN, H), lambda p, j: (j, 0)),
            pl.BlockSpec((BN, H), lambda p, j: (j, 0)),
            pl.BlockSpec((1, 1, BN), lambda p, j: (j, 0, 0)),
            pl.BlockSpec((1, D), lambda p, j: (0, 0)),
        ],
        out_specs=pl.BlockSpec((G, D), lambda p, j: (0, 0)),
        out_shape=jax.ShapeDtypeStruct((G, D), jnp.float32),
        scratch_shapes=[
            pltpu.VMEM((G, 128), jnp.float32),
            pltpu.VMEM((G, 128), jnp.float32),
            pltpu.VMEM((G, D), jnp.float32),
        ],
    )(a0, a1, batch3, gw)


def _sc_agg_body(ht0, ht1, src_hbm, dst_hbm, zeros_hbm, o0, o1,
                 src_v, dst_v, rows, gsems, ssems, acc):
    c = lax.axis_index("c")
    s = lax.axis_index("s")

    pltpu.sync_copy(zeros_hbm, acc.at[pl.ds(s * ROWS_PER_TILE, ROWS_PER_TILE)])

    def run(table, out_ref):
        plsc.subcore_barrier()

        def wait_gather(b):
            pltpu.make_async_copy(table.at[src_v.at[0]], rows[b],
                                  gsems[b]).wait()

        def wait_scatter(b):
            pltpu.make_async_copy(rows[b], acc.at[dst_v.at[0]],
                                  ssems[b]).wait()

        @pl.loop(0, NBLK)
        def _(k):
            blk = s * CPT + k * IB
            pltpu.sync_copy(src_hbm.at[pl.ds(blk, IB)], src_v)
            pltpu.sync_copy(dst_hbm.at[pl.ds(blk, IB)], dst_v)
            for b in range(NBUF):
                pltpu.async_copy(table.at[src_v.at[b]], rows[b], gsems[b])

            @pl.loop(0, IB // NBUF)
            def _(r):
                j = r * NBUF
                for b in range(NBUF):
                    wait_gather(b)
                    pltpu.async_copy(rows[b], acc.at[dst_v.at[j + b]],
                                     ssems[b], add=True)
                for b in range(NBUF):
                    @pl.when(j + b + NBUF < IB)
                    def _(b=b):
                        wait_scatter(b)
                        pltpu.async_copy(table.at[src_v.at[j + b + NBUF]],
                                         rows[b], gsems[b])

            for b in range(NBUF):
                wait_scatter(b)

        plsc.subcore_barrier()
        base = s * ROWS_PER_TILE
        pltpu.sync_copy(acc.at[pl.ds(base, ROWS_PER_TILE)],
                        out_ref.at[pl.ds(base, ROWS_PER_TILE)])

    @pl.when(c == 0)
    def _():
        run(ht0, o0)

    @pl.when(c == 1)
    def _():
        run(ht1, o1)


@functools.cache
def _make_sc_agg():
    return pl.kernel(
        _sc_agg_body,
        out_type=[jax.ShapeDtypeStruct((NP, H), jnp.float32)] * 2,
        mesh=plsc.VectorSubcoreMesh(core_axis_name="c", subcore_axis_name="s"),
        scratch_types=[
            pltpu.VMEM((IB, CHUNK), jnp.int32),
            pltpu.VMEM((IB, CHUNK), jnp.int32),
            [pltpu.VMEM((CHUNK, H), jnp.float32) for _ in range(NBUF)],
            [pltpu.SemaphoreType.DMA for _ in range(NBUF)],
            [pltpu.SemaphoreType.DMA for _ in range(NBUF)],
            pltpu.VMEM_SHARED((NACC, H), jnp.float32),
        ],
    )


def _sc_agg(ht0, ht1, src2d, dst2d, zeros):
    return _make_sc_agg()(ht0, ht1, src2d, dst2d, zeros)


def kernel(x, edge_index, batch, W1, b1, W2, b2, gate_w, gate_b):
    n = x.shape[0]
    e = edge_index.shape[1]

    xp = jnp.zeros((NP, D), jnp.float32).at[:n].set(x)
    batchp = jnp.full((NP,), G, jnp.int32).at[:n].set(batch)
    batch3 = batchp.reshape(NB, 1, BN)

    pad = EP - e
    filler = jnp.arange(pad, dtype=jnp.int32)
    src = jnp.concatenate([edge_index[0], filler % n]).reshape(NSUB * CPT, CHUNK)
    dst = jnp.concatenate([edge_index[1], NP + (filler % 8)]).reshape(NSUB * CPT, CHUNK)
    zeros = jnp.zeros((ROWS_PER_TILE, H), jnp.float32)

    b1r = b1.reshape(1, D)
    b2r = b2.reshape(1, D)
    gw = gate_w.reshape(1, D)
    del gate_b

    ht0, ht1 = _tc_in(xp, W1, b1r)
    a0, a1 = _sc_agg(ht0, ht1, src, dst, zeros)
    ht0, ht1 = _tc_mid(a0, a1, W2, b2r)
    a0, a1 = _sc_agg(ht0, ht1, src, dst, zeros)
    return _tc_pool(a0, a1, batch3, gw)


# speedup vs baseline: 1.2679x; 1.2679x over previous
import functools

import jax
import jax.numpy as jnp
from jax import lax
from jax.experimental import pallas as pl
from jax.experimental.pallas import tpu as pltpu
from jax.experimental.pallas import tpu_sc as plsc

NN = 10000
D = 256
H = 128
G = 64
BN = 1000
NB = NN // BN
NSUB = 16
CHUNK = 128
CPT = 80
IB = 16
NBLK = CPT // IB
EP = NSUB * CPT * CHUNK
ROWS_PER_TILE = 632
NSC = NSUB * ROWS_PER_TILE
NACC = NSC

_MAXNORM = 1.0 - 4e-3
_EPS = 1e-15


def _rnorm(x):
    return jnp.maximum(jnp.sqrt(jnp.sum(x * x, axis=-1, keepdims=True)), _EPS)


def _artanh(x):
    x = jnp.clip(x, -1.0 + 1e-7, 1.0 - 1e-7)
    return 0.5 * jnp.log((1.0 + x) / (1.0 - x))


def _proj(x):
    n = _rnorm(x)
    return x * jnp.minimum(1.0, _MAXNORM / n)


def _proj_expmap0(u):
    un = _rnorm(u)
    return u * (jnp.minimum(jnp.tanh(un), _MAXNORM) / un)


def _logmap0(p):
    pn = _rnorm(p)
    return p * (_artanh(pn) / pn)


def _proj_mobius_add(x, y):
    x2 = jnp.sum(x * x, -1, keepdims=True)
    y2 = jnp.sum(y * y, -1, keepdims=True)
    xy = jnp.sum(x * y, -1, keepdims=True)
    num = (1.0 + 2.0 * xy + y2) * x + (1.0 - x2) * y
    den = jnp.maximum(1.0 + 2.0 * xy + x2 * y2, _EPS)
    nn = _rnorm(num)
    return num * jnp.minimum(1.0 / den, _MAXNORM / nn)


def _proj_mobius_matvec(w, x):
    xn = _rnorm(x)
    mx = lax.dot_general(x, w, (((1,), (1,)), ((), ())),
                         preferred_element_type=jnp.float32)
    mxn = _rnorm(mx)
    return mx * (jnp.minimum(jnp.tanh(mxn / xn * _artanh(xn)), _MAXNORM) / mxn)


def _hyp_linear(w, b, h):
    mv = _proj_mobius_matvec(w, h)
    hb = _proj_expmap0(b)
    return _proj_mobius_add(mv, hb)


def _post_agg(agg):
    h = _proj_expmap0(agg)
    ht = jax.nn.relu(_logmap0(h))
    return _proj_expmap0(ht)


def _tc_in_body(x_ref, w_ref, b_ref, o0_ref, o1_ref):
    h = _proj_expmap0(x_ref[...])
    h = _hyp_linear(w_ref[...], b_ref[...], h)
    ht = _logmap0(h)
    o0_ref[...] = ht[:, :H]
    o1_ref[...] = ht[:, H:]


def _tc_in(xp, w, b):
    return pl.pallas_call(
        _tc_in_body,
        grid=(NB,),
        in_specs=[
            pl.BlockSpec((BN, D), lambda i: (i, 0)),
            pl.BlockSpec((D, D), lambda i: (0, 0)),
            pl.BlockSpec((1, D), lambda i: (0, 0)),
        ],
        out_specs=[
            pl.BlockSpec((BN, H), lambda i: (i, 0)),
            pl.BlockSpec((BN, H), lambda i: (i, 0)),
        ],
        out_shape=[jax.ShapeDtypeStruct((NN, H), jnp.float32)] * 2,
    )(xp, w, b)


def _tc_mid_body(a0_ref, a1_ref, w_ref, b_ref, o0_ref, o1_ref):
    agg = jnp.concatenate([a0_ref[...], a1_ref[...]], axis=1)
    h = _post_agg(agg)
    h = _hyp_linear(w_ref[...], b_ref[...], h)
    ht = _logmap0(h)
    o0_ref[...] = ht[:, :H]
    o1_ref[...] = ht[:, H:]


def _tc_mid(a0, a1, w, b):
    return pl.pallas_call(
        _tc_mid_body,
        grid=(NB,),
        in_specs=[
            pl.BlockSpec((BN, H), lambda i: (i, 0)),
            pl.BlockSpec((BN, H), lambda i: (i, 0)),
            pl.BlockSpec((D, D), lambda i: (0, 0)),
            pl.BlockSpec((1, D), lambda i: (0, 0)),
        ],
        out_specs=[
            pl.BlockSpec((BN, H), lambda i: (i, 0)),
            pl.BlockSpec((BN, H), lambda i: (i, 0)),
        ],
        out_shape=[jax.ShapeDtypeStruct((NN, H), jnp.float32)] * 2,
    )(a0, a1, w, b)


def _tc_pool_body(a0_ref, a1_ref, batch_ref, gw_ref, out_ref,
                  smax_s, den_s, num_s):
    j = pl.program_id(0)

    agg = jnp.concatenate([a0_ref[...], a1_ref[...]], axis=1)
    h = _post_agg(agg)
    gw = gw_ref[...]
    gl = lax.dot_general(gw, h, (((1,), (1,)), ((), ())),
                         preferred_element_type=jnp.float32)
    b2d = batch_ref[...].reshape(1, BN)
    seg = lax.broadcasted_iota(jnp.int32, (G, BN), 0)
    mask = seg == b2d

    @pl.when(j == 0)
    def _():
        smax_s[...] = jnp.full_like(smax_s[...], -1e30)
        den_s[...] = jnp.zeros_like(den_s[...])
        num_s[...] = jnp.zeros_like(num_s[...])

    bm = jnp.max(jnp.where(mask, gl, -1e30), axis=1, keepdims=True)
    m_old = jnp.max(smax_s[...], axis=1, keepdims=True)
    m_new = jnp.maximum(m_old, bm)
    scale = jnp.exp(m_old - m_new)
    e = jnp.where(mask, jnp.exp(gl - m_new), 0.0)
    smax_s[...] = jnp.broadcast_to(m_new, smax_s.shape)
    den_s[...] = den_s[...] * scale + jnp.sum(e, axis=1, keepdims=True)
    num_s[...] = num_s[...] * scale + lax.dot_general(
        e, h, (((1,), (0,)), ((), ())), preferred_element_type=jnp.float32)

    @pl.when(j == NB - 1)
    def _():
        den = jnp.max(den_s[...], axis=1, keepdims=True)
        out_ref[...] = num_s[...] / (den + 1e-16)


def _tc_pool(a0, a1, batch3, gw):
    return pl.pallas_call(
        _tc_pool_body,
        grid=(NB,),
        in_specs=[
            pl.BlockSpec((BN, H), lambda j: (j, 0)),
            pl.BlockSpec((BN, H), lambda j: (j, 0)),
            pl.BlockSpec((1, 1, BN), lambda j: (j, 0, 0)),
            pl.BlockSpec((1, D), lambda j: (0, 0)),
        ],
        out_specs=pl.BlockSpec((G, D), lambda j: (0, 0)),
        out_shape=jax.ShapeDtypeStruct((G, D), jnp.float32),
        scratch_shapes=[
            pltpu.VMEM((G, 128), jnp.float32),
            pltpu.VMEM((G, 128), jnp.float32),
            pltpu.VMEM((G, D), jnp.float32),
        ],
    )(a0, a1, batch3, gw)


def _sc_agg_body(ht0, ht1, src_hbm, dst_hbm, zeros_hbm, o0, o1,
                 src_v, dst_v, rows0, rows1, acc, sem0, sem1):
    c = lax.axis_index("c")
    s = lax.axis_index("s")

    pltpu.sync_copy(zeros_hbm, acc.at[pl.ds(s * ROWS_PER_TILE, ROWS_PER_TILE)])

    def run(table, out_ref):
        plsc.subcore_barrier()

        @pl.loop(0, NBLK)
        def _(k):
            blk = s * CPT + k * IB
            pltpu.sync_copy(src_hbm.at[pl.ds(blk, IB)], src_v)
            pltpu.sync_copy(dst_hbm.at[pl.ds(blk, IB)], dst_v)
            pltpu.async_copy(table.at[src_v.at[0]], rows0, sem0)

            @pl.loop(0, IB // 2)
            def _(i):
                j0 = 2 * i
                pltpu.async_copy(table.at[src_v.at[j0 + 1]], rows1, sem1)
                pltpu.make_async_copy(table.at[src_v.at[j0]], rows0, sem0).wait()
                pltpu.sync_copy(rows0, acc.at[dst_v.at[j0]], add=True)

                @pl.when(j0 + 2 < IB)
                def _():
                    pltpu.async_copy(table.at[src_v.at[j0 + 2]], rows0, sem0)

                pltpu.make_async_copy(table.at[src_v.at[j0 + 1]], rows1, sem1).wait()
                pltpu.sync_copy(rows1, acc.at[dst_v.at[j0 + 1]], add=True)

        plsc.subcore_barrier()
        base = s * ROWS_PER_TILE
        pltpu.sync_copy(acc.at[pl.ds(base, ROWS_PER_TILE)],
                        out_ref.at[pl.ds(base, ROWS_PER_TILE)])

    @pl.when(c == 0)
    def _():
        run(ht0, o0)

    @pl.when(c == 1)
    def _():
        run(ht1, o1)


@functools.cache
def _make_sc_agg():
    return pl.kernel(
        _sc_agg_body,
        out_type=[jax.ShapeDtypeStruct((NSC, H), jnp.float32)] * 2,
        mesh=plsc.VectorSubcoreMesh(core_axis_name="c", subcore_axis_name="s"),
        scratch_types=[
            pltpu.VMEM((IB, CHUNK), jnp.int32),
            pltpu.VMEM((IB, CHUNK), jnp.int32),
            pltpu.VMEM((CHUNK, H), jnp.float32),
            pltpu.VMEM((CHUNK, H), jnp.float32),
            pltpu.VMEM_SHARED((NACC, H), jnp.float32),
            pltpu.SemaphoreType.DMA,
            pltpu.SemaphoreType.DMA,
        ],
    )


def _sc_agg(ht0, ht1, src2d, dst2d, zeros):
    return _make_sc_agg()(ht0, ht1, src2d, dst2d, zeros)


def kernel(x, edge_index, batch, W1, b1, W2, b2, gate_w, gate_b):
    n = x.shape[0]
    e = edge_index.shape[1]

    batch3 = batch.astype(jnp.int32).reshape(NB, 1, BN)

    pad = EP - e
    filler = jnp.arange(pad, dtype=jnp.int32)
    src = jnp.concatenate([edge_index[0], filler % n]).reshape(NSUB * CPT, CHUNK)
    dst = jnp.concatenate([edge_index[1], n + (filler % 8)]).reshape(NSUB * CPT, CHUNK)
    zeros = jnp.zeros((ROWS_PER_TILE, H), jnp.float32)

    b1r = b1.reshape(1, D)
    b2r = b2.reshape(1, D)
    gw = gate_w.reshape(1, D)
    del gate_b

    ht0, ht1 = _tc_in(x, W1, b1r)
    a0, a1 = _sc_agg(ht0, ht1, src, dst, zeros)
    ht0, ht1 = _tc_mid(a0, a1, W2, b2r)
    a0, a1 = _sc_agg(ht0, ht1, src, dst, zeros)
    return _tc_pool(a0, a1, batch3, gw)


# speedup vs baseline: 1.3100x; 1.0332x over previous
import functools

import jax
import jax.numpy as jnp
from jax import lax
from jax.experimental import pallas as pl
from jax.experimental.pallas import tpu as pltpu
from jax.experimental.pallas import tpu_sc as plsc

NN = 10000
D = 256
H = 128
G = 64
BN = 1000
NB = NN // BN
NSUB = 16
CHUNK = 128
CPT = 80
IB = 16
NBLK = CPT // IB
EP = NSUB * CPT * CHUNK
ROWS_PER_TILE = 632
NSC = NSUB * ROWS_PER_TILE
NACC = NSC

_MAXNORM = 1.0 - 4e-3
_EPS = 1e-15


def _rnorm(x):
    return jnp.maximum(jnp.sqrt(jnp.sum(x * x, axis=-1, keepdims=True)), _EPS)


def _artanh(x):
    x = jnp.clip(x, -1.0 + 1e-7, 1.0 - 1e-7)
    return 0.5 * jnp.log((1.0 + x) / (1.0 - x))


def _proj(x):
    n = _rnorm(x)
    return x * jnp.minimum(1.0, _MAXNORM / n)


def _proj_expmap0(u):
    un = _rnorm(u)
    return u * (jnp.minimum(jnp.tanh(un), _MAXNORM) / un)


def _logmap0(p):
    pn = _rnorm(p)
    return p * (_artanh(pn) / pn)


def _proj_expmap0_n(u):
    un = _rnorm(u)
    t = jnp.minimum(jnp.tanh(un), _MAXNORM)
    return u * (t / un), jnp.maximum(t, _EPS)


def _linear_logmap(w, b, h, hn):
    mx = lax.dot_general(h, w, (((1,), (1,)), ((), ())),
                         preferred_element_type=jnp.float32)
    mxn = _rnorm(mx)
    al = jnp.minimum(jnp.tanh(mxn / hn * _artanh(hn)), _MAXNORM)
    mv = mx * (al / mxn)
    x2 = al * al
    hb, _ = _proj_expmap0_n(b)
    y2 = jnp.sum(hb * hb, -1, keepdims=True)
    xy = jnp.sum(mv * hb, -1, keepdims=True)
    num = (1.0 + 2.0 * xy + y2) * mv + (1.0 - x2) * hb
    den = jnp.maximum(1.0 + 2.0 * xy + x2 * y2, _EPS)
    nn = _rnorm(num)
    rn = jnp.maximum(jnp.minimum(nn / den, _MAXNORM), _EPS)
    return num * (jnp.minimum(1.0 / den, _MAXNORM / nn) * (_artanh(rn) / rn))


def _post_agg(agg):
    n1 = _rnorm(agg)
    t1 = jnp.minimum(jnp.tanh(n1), _MAXNORM)
    t1c = jnp.maximum(t1, _EPS)
    ht = jax.nn.relu(agg * ((t1 / n1) * (_artanh(t1c) / t1c)))
    return _proj_expmap0_n(ht)


def _tc_in_body(x_ref, w_ref, b_ref, o0_ref, o1_ref):
    h, hn = _proj_expmap0_n(x_ref[...])
    ht = _linear_logmap(w_ref[...], b_ref[...], h, hn)
    o0_ref[...] = ht[:, :H]
    o1_ref[...] = ht[:, H:]


def _tc_in(xp, w, b):
    return pl.pallas_call(
        _tc_in_body,
        grid=(NB,),
        in_specs=[
            pl.BlockSpec((BN, D), lambda i: (i, 0)),
            pl.BlockSpec((D, D), lambda i: (0, 0)),
            pl.BlockSpec((1, D), lambda i: (0, 0)),
        ],
        out_specs=[
            pl.BlockSpec((BN, H), lambda i: (i, 0)),
            pl.BlockSpec((BN, H), lambda i: (i, 0)),
        ],
        out_shape=[jax.ShapeDtypeStruct((NN, H), jnp.float32)] * 2,
    )(xp, w, b)


def _tc_mid_body(a0_ref, a1_ref, w_ref, b_ref, o0_ref, o1_ref):
    agg = jnp.concatenate([a0_ref[...], a1_ref[...]], axis=1)
    h, hn = _post_agg(agg)
    ht = _linear_logmap(w_ref[...], b_ref[...], h, hn)
    o0_ref[...] = ht[:, :H]
    o1_ref[...] = ht[:, H:]


def _tc_mid(a0, a1, w, b):
    return pl.pallas_call(
        _tc_mid_body,
        grid=(NB,),
        in_specs=[
            pl.BlockSpec((BN, H), lambda i: (i, 0)),
            pl.BlockSpec((BN, H), lambda i: (i, 0)),
            pl.BlockSpec((D, D), lambda i: (0, 0)),
            pl.BlockSpec((1, D), lambda i: (0, 0)),
        ],
        out_specs=[
            pl.BlockSpec((BN, H), lambda i: (i, 0)),
            pl.BlockSpec((BN, H), lambda i: (i, 0)),
        ],
        out_shape=[jax.ShapeDtypeStruct((NN, H), jnp.float32)] * 2,
    )(a0, a1, w, b)


def _tc_pool_body(a0_ref, a1_ref, batch_ref, gw_ref, out_ref,
                  smax_s, den_s, num_s):
    j = pl.program_id(0)

    agg = jnp.concatenate([a0_ref[...], a1_ref[...]], axis=1)
    h, _ = _post_agg(agg)
    gw = gw_ref[...]
    gl = lax.dot_general(gw, h, (((1,), (1,)), ((), ())),
                         preferred_element_type=jnp.float32)
    b2d = batch_ref[...].reshape(1, BN)
    seg = lax.broadcasted_iota(jnp.int32, (G, BN), 0)
    mask = seg == b2d

    @pl.when(j == 0)
    def _():
        smax_s[...] = jnp.full_like(smax_s[...], -1e30)
        den_s[...] = jnp.zeros_like(den_s[...])
        num_s[...] = jnp.zeros_like(num_s[...])

    bm = jnp.max(jnp.where(mask, gl, -1e30), axis=1, keepdims=True)
    m_old = jnp.max(smax_s[...], axis=1, keepdims=True)
    m_new = jnp.maximum(m_old, bm)
    scale = jnp.exp(m_old - m_new)
    e = jnp.where(mask, jnp.exp(gl - m_new), 0.0)
    smax_s[...] = jnp.broadcast_to(m_new, smax_s.shape)
    den_s[...] = den_s[...] * scale + jnp.sum(e, axis=1, keepdims=True)
    num_s[...] = num_s[...] * scale + lax.dot_general(
        e, h, (((1,), (0,)), ((), ())), preferred_element_type=jnp.float32)

    @pl.when(j == NB - 1)
    def _():
        den = jnp.max(den_s[...], axis=1, keepdims=True)
        out_ref[...] = num_s[...] / (den + 1e-16)


def _tc_pool(a0, a1, batch3, gw):
    return pl.pallas_call(
        _tc_pool_body,
        grid=(NB,),
        in_specs=[
            pl.BlockSpec((BN, H), lambda j: (j, 0)),
            pl.BlockSpec((BN, H), lambda j: (j, 0)),
            pl.BlockSpec((1, 1, BN), lambda j: (j, 0, 0)),
            pl.BlockSpec((1, D), lambda j: (0, 0)),
        ],
        out_specs=pl.BlockSpec((G, D), lambda j: (0, 0)),
        out_shape=jax.ShapeDtypeStruct((G, D), jnp.float32),
        scratch_shapes=[
            pltpu.VMEM((G, 128), jnp.float32),
            pltpu.VMEM((G, 128), jnp.float32),
            pltpu.VMEM((G, D), jnp.float32),
        ],
    )(a0, a1, batch3, gw)


def _sc_agg_body(ht0, ht1, src_hbm, dst_hbm, zeros_hbm, o0, o1,
                 src_v, dst_v, rows0, rows1, acc, sem0, sem1):
    c = lax.axis_index("c")
    s = lax.axis_index("s")

    pltpu.sync_copy(zeros_hbm, acc.at[pl.ds(s * ROWS_PER_TILE, ROWS_PER_TILE)])

    def run(table, out_ref):
        plsc.subcore_barrier()

        @pl.loop(0, NBLK)
        def _(k):
            blk = s * CPT + k * IB
            pltpu.sync_copy(src_hbm.at[pl.ds(blk, IB)], src_v)
            pltpu.sync_copy(dst_hbm.at[pl.ds(blk, IB)], dst_v)
            pltpu.async_copy(table.at[src_v.at[0]], rows0, sem0)

            @pl.loop(0, IB // 2)
            def _(i):
                j0 = 2 * i
                pltpu.async_copy(table.at[src_v.at[j0 + 1]], rows1, sem1)
                pltpu.make_async_copy(table.at[src_v.at[j0]], rows0, sem0).wait()
                pltpu.sync_copy(rows0, acc.at[dst_v.at[j0]], add=True)

                @pl.when(j0 + 2 < IB)
                def _():
                    pltpu.async_copy(table.at[src_v.at[j0 + 2]], rows0, sem0)

                pltpu.make_async_copy(table.at[src_v.at[j0 + 1]], rows1, sem1).wait()
                pltpu.sync_copy(rows1, acc.at[dst_v.at[j0 + 1]], add=True)

        plsc.subcore_barrier()
        base = s * ROWS_PER_TILE
        pltpu.sync_copy(acc.at[pl.ds(base, ROWS_PER_TILE)],
                        out_ref.at[pl.ds(base, ROWS_PER_TILE)])

    @pl.when(c == 0)
    def _():
        run(ht0, o0)

    @pl.when(c == 1)
    def _():
        run(ht1, o1)


@functools.cache
def _make_sc_agg():
    return pl.kernel(
        _sc_agg_body,
        out_type=[jax.ShapeDtypeStruct((NSC, H), jnp.float32)] * 2,
        mesh=plsc.VectorSubcoreMesh(core_axis_name="c", subcore_axis_name="s"),
        scratch_types=[
            pltpu.VMEM((IB, CHUNK), jnp.int32),
            pltpu.VMEM((IB, CHUNK), jnp.int32),
            pltpu.VMEM((CHUNK, H), jnp.float32),
            pltpu.VMEM((CHUNK, H), jnp.float32),
            pltpu.VMEM_SHARED((NACC, H), jnp.float32),
            pltpu.SemaphoreType.DMA,
            pltpu.SemaphoreType.DMA,
        ],
    )


def _sc_agg(ht0, ht1, src2d, dst2d, zeros):
    return _make_sc_agg()(ht0, ht1, src2d, dst2d, zeros)


def kernel(x, edge_index, batch, W1, b1, W2, b2, gate_w, gate_b):
    n = x.shape[0]
    e = edge_index.shape[1]

    batch3 = batch.astype(jnp.int32).reshape(NB, 1, BN)

    pad = EP - e
    filler = jnp.arange(pad, dtype=jnp.int32)
    src = jnp.concatenate([edge_index[0], filler % n]).reshape(NSUB * CPT, CHUNK)
    dst = jnp.concatenate([edge_index[1], n + (filler % 8)]).reshape(NSUB * CPT, CHUNK)
    zeros = jnp.zeros((ROWS_PER_TILE, H), jnp.float32)

    b1r = b1.reshape(1, D)
    b2r = b2.reshape(1, D)
    gw = gate_w.reshape(1, D)
    del gate_b

    ht0, ht1 = _tc_in(x, W1, b1r)
    a0, a1 = _sc_agg(ht0, ht1, src, dst, zeros)
    ht0, ht1 = _tc_mid(a0, a1, W2, b2r)
    a0, a1 = _sc_agg(ht0, ht1, src, dst, zeros)
    return _tc_pool(a0, a1, batch3, gw)


# speedup vs baseline: 1.3642x; 1.0414x over previous
import functools

import jax
import jax.numpy as jnp
from jax import lax
from jax.experimental import pallas as pl
from jax.experimental.pallas import tpu as pltpu
from jax.experimental.pallas import tpu_sc as plsc

NN = 10000
D = 256
H = 128
G = 64
BN = 1000
NB = NN // BN
NSUB = 16
CHUNK = 128
CPT = 80
IB = 16
NBLK = CPT // IB
EP = NSUB * CPT * CHUNK
ROWS_PER_TILE = 632
NSC = NSUB * ROWS_PER_TILE
NACC = NSC

_MAXNORM = 1.0 - 4e-3
_EPS = 1e-15


def _rnorm(x):
    return jnp.maximum(jnp.sqrt(jnp.sum(x * x, axis=-1, keepdims=True)), _EPS)


def _artanh(x):
    x = jnp.clip(x, -1.0 + 1e-7, 1.0 - 1e-7)
    return 0.5 * jnp.log((1.0 + x) / (1.0 - x))


def _proj(x):
    n = _rnorm(x)
    return x * jnp.minimum(1.0, _MAXNORM / n)


def _proj_expmap0(u):
    un = _rnorm(u)
    return u * (jnp.minimum(jnp.tanh(un), _MAXNORM) / un)


def _logmap0(p):
    pn = _rnorm(p)
    return p * (_artanh(pn) / pn)


def _proj_expmap0_n(u):
    un = _rnorm(u)
    t = jnp.minimum(jnp.tanh(un), _MAXNORM)
    return u * (t / un), jnp.maximum(t, _EPS)


def _linear_logmap(w, b, h, hn):
    mx = lax.dot_general(h, w, (((1,), (1,)), ((), ())),
                         preferred_element_type=jnp.float32)
    mxn = _rnorm(mx)
    al = jnp.minimum(jnp.tanh(mxn / hn * _artanh(hn)), _MAXNORM)
    mv = mx * (al / mxn)
    x2 = al * al
    hb, _ = _proj_expmap0_n(b)
    y2 = jnp.sum(hb * hb, -1, keepdims=True)
    xy = jnp.sum(mv * hb, -1, keepdims=True)
    num = (1.0 + 2.0 * xy + y2) * mv + (1.0 - x2) * hb
    den = jnp.maximum(1.0 + 2.0 * xy + x2 * y2, _EPS)
    nn = _rnorm(num)
    rn = jnp.maximum(jnp.minimum(nn / den, _MAXNORM), _EPS)
    return num * (jnp.minimum(1.0 / den, _MAXNORM / nn) * (_artanh(rn) / rn))


def _post_agg(agg):
    n1 = _rnorm(agg)
    t1 = jnp.minimum(jnp.tanh(n1), _MAXNORM)
    t1c = jnp.maximum(t1, _EPS)
    ht = jax.nn.relu(agg * ((t1 / n1) * (_artanh(t1c) / t1c)))
    return _proj_expmap0_n(ht)


def _tc_in_body(x_ref, w_ref, b_ref, o0_ref, o1_ref):
    h, hn = _proj_expmap0_n(x_ref[...])
    ht = _linear_logmap(w_ref[...], b_ref[...], h, hn)
    o0_ref[...] = ht[:, :H]
    o1_ref[...] = ht[:, H:]


def _tc_in(xp, w, b):
    return pl.pallas_call(
        _tc_in_body,
        grid=(NB,),
        in_specs=[
            pl.BlockSpec((BN, D), lambda i: (i, 0)),
            pl.BlockSpec((D, D), lambda i: (0, 0)),
            pl.BlockSpec((1, D), lambda i: (0, 0)),
        ],
        out_specs=[
            pl.BlockSpec((BN, H), lambda i: (i, 0)),
            pl.BlockSpec((BN, H), lambda i: (i, 0)),
        ],
        out_shape=[jax.ShapeDtypeStruct((NN, H), jnp.float32)] * 2,
    )(xp, w, b)


def _tc_mid_body(a0_ref, a1_ref, w_ref, b_ref, o0_ref, o1_ref):
    agg = jnp.concatenate([a0_ref[...], a1_ref[...]], axis=1)
    h, hn = _post_agg(agg)
    ht = _linear_logmap(w_ref[...], b_ref[...], h, hn)
    o0_ref[...] = ht[:, :H]
    o1_ref[...] = ht[:, H:]


def _tc_mid(a0, a1, w, b):
    return pl.pallas_call(
        _tc_mid_body,
        grid=(NB,),
        in_specs=[
            pl.BlockSpec((BN, H), lambda i: (i, 0)),
            pl.BlockSpec((BN, H), lambda i: (i, 0)),
            pl.BlockSpec((D, D), lambda i: (0, 0)),
            pl.BlockSpec((1, D), lambda i: (0, 0)),
        ],
        out_specs=[
            pl.BlockSpec((BN, H), lambda i: (i, 0)),
            pl.BlockSpec((BN, H), lambda i: (i, 0)),
        ],
        out_shape=[jax.ShapeDtypeStruct((NN, H), jnp.float32)] * 2,
    )(a0, a1, w, b)


def _tc_pool_body(a0_ref, a1_ref, batch_ref, gw_ref, out_ref,
                  smax_s, den_s, num_s):
    j = pl.program_id(0)

    agg = jnp.concatenate([a0_ref[...], a1_ref[...]], axis=1)
    h, _ = _post_agg(agg)
    gw = gw_ref[...]
    gl = lax.dot_general(gw, h, (((1,), (1,)), ((), ())),
                         preferred_element_type=jnp.float32)
    b2d = batch_ref[...].reshape(1, BN)
    seg = lax.broadcasted_iota(jnp.int32, (G, BN), 0)
    mask = seg == b2d

    @pl.when(j == 0)
    def _():
        smax_s[...] = jnp.full_like(smax_s[...], -1e30)
        den_s[...] = jnp.zeros_like(den_s[...])
        num_s[...] = jnp.zeros_like(num_s[...])

    bm = jnp.max(jnp.where(mask, gl, -1e30), axis=1, keepdims=True)
    m_old = jnp.max(smax_s[...], axis=1, keepdims=True)
    m_new = jnp.maximum(m_old, bm)
    scale = jnp.exp(m_old - m_new)
    e = jnp.where(mask, jnp.exp(gl - m_new), 0.0)
    smax_s[...] = jnp.broadcast_to(m_new, smax_s.shape)
    den_s[...] = den_s[...] * scale + jnp.sum(e, axis=1, keepdims=True)
    num_s[...] = num_s[...] * scale + lax.dot_general(
        e, h, (((1,), (0,)), ((), ())), preferred_element_type=jnp.float32)

    @pl.when(j == NB - 1)
    def _():
        den = jnp.max(den_s[...], axis=1, keepdims=True)
        out_ref[...] = num_s[...] / (den + 1e-16)


def _tc_pool(a0, a1, batch3, gw):
    return pl.pallas_call(
        _tc_pool_body,
        grid=(NB,),
        in_specs=[
            pl.BlockSpec((BN, H), lambda j: (j, 0)),
            pl.BlockSpec((BN, H), lambda j: (j, 0)),
            pl.BlockSpec((1, 1, BN), lambda j: (j, 0, 0)),
            pl.BlockSpec((1, D), lambda j: (0, 0)),
        ],
        out_specs=pl.BlockSpec((G, D), lambda j: (0, 0)),
        out_shape=jax.ShapeDtypeStruct((G, D), jnp.float32),
        scratch_shapes=[
            pltpu.VMEM((G, 128), jnp.float32),
            pltpu.VMEM((G, 128), jnp.float32),
            pltpu.VMEM((G, D), jnp.float32),
        ],
    )(a0, a1, batch3, gw)


def _sc_agg_body(ht0, ht1, src_hbm, dst_hbm, zeros_hbm, o0, o1,
                 src_bufs, dst_bufs, rows0, rows1, acc, sem0, sem1, isem):
    c = lax.axis_index("c")
    s = lax.axis_index("s")

    pltpu.sync_copy(zeros_hbm, acc.at[pl.ds(s * ROWS_PER_TILE, ROWS_PER_TILE)])

    def run(table, out_ref):
        base = s * CPT
        pltpu.sync_copy(src_hbm.at[pl.ds(base, IB)], src_bufs[0])
        pltpu.sync_copy(dst_hbm.at[pl.ds(base, IB)], dst_bufs[0])
        pltpu.async_copy(table.at[src_bufs[0].at[0]], rows0, sem0)
        plsc.subcore_barrier()

        for k in range(NBLK):
            src_v, dst_v = src_bufs[k % 2], dst_bufs[k % 2]
            if k + 1 < NBLK:
                nsrc, ndst = src_bufs[(k + 1) % 2], dst_bufs[(k + 1) % 2]
                pltpu.async_copy(src_hbm.at[pl.ds(base + (k + 1) * IB, IB)],
                                 nsrc, isem)
                pltpu.async_copy(dst_hbm.at[pl.ds(base + (k + 1) * IB, IB)],
                                 ndst, isem)

            @pl.loop(0, IB // 2)
            def _(i):
                j0 = 2 * i
                pltpu.async_copy(table.at[src_v.at[j0 + 1]], rows1, sem1)
                pltpu.make_async_copy(table.at[src_v.at[j0]], rows0, sem0).wait()
                pltpu.sync_copy(rows0, acc.at[dst_v.at[j0]], add=True)

                @pl.when(j0 + 2 < IB)
                def _():
                    pltpu.async_copy(table.at[src_v.at[j0 + 2]], rows0, sem0)

                pltpu.make_async_copy(table.at[src_v.at[j0 + 1]], rows1, sem1).wait()
                pltpu.sync_copy(rows1, acc.at[dst_v.at[j0 + 1]], add=True)

            if k + 1 < NBLK:
                pltpu.make_async_copy(src_hbm.at[pl.ds(0, IB)], nsrc, isem).wait()
                pltpu.make_async_copy(dst_hbm.at[pl.ds(0, IB)], ndst, isem).wait()
                pltpu.async_copy(table.at[nsrc.at[0]], rows0, sem0)

        plsc.subcore_barrier()
        wb = s * ROWS_PER_TILE
        pltpu.sync_copy(acc.at[pl.ds(wb, ROWS_PER_TILE)],
                        out_ref.at[pl.ds(wb, ROWS_PER_TILE)])

    @pl.when(c == 0)
    def _():
        run(ht0, o0)

    @pl.when(c == 1)
    def _():
        run(ht1, o1)


@functools.cache
def _make_sc_agg():
    return pl.kernel(
        _sc_agg_body,
        out_type=[jax.ShapeDtypeStruct((NSC, H), jnp.float32)] * 2,
        mesh=plsc.VectorSubcoreMesh(core_axis_name="c", subcore_axis_name="s"),
        scratch_types=[
            [pltpu.VMEM((IB, CHUNK), jnp.int32) for _ in range(2)],
            [pltpu.VMEM((IB, CHUNK), jnp.int32) for _ in range(2)],
            pltpu.VMEM((CHUNK, H), jnp.float32),
            pltpu.VMEM((CHUNK, H), jnp.float32),
            pltpu.VMEM_SHARED((NACC, H), jnp.float32),
            pltpu.SemaphoreType.DMA,
            pltpu.SemaphoreType.DMA,
            pltpu.SemaphoreType.DMA,
        ],
    )


def _sc_agg(ht0, ht1, src2d, dst2d, zeros):
    return _make_sc_agg()(ht0, ht1, src2d, dst2d, zeros)


def kernel(x, edge_index, batch, W1, b1, W2, b2, gate_w, gate_b):
    n = x.shape[0]
    e = edge_index.shape[1]

    batch3 = batch.astype(jnp.int32).reshape(NB, 1, BN)

    pad = EP - e
    filler = jnp.arange(pad, dtype=jnp.int32)
    src = jnp.concatenate([edge_index[0], filler % n]).reshape(NSUB * CPT, CHUNK)
    dst = jnp.concatenate([edge_index[1], n + (filler % 8)]).reshape(NSUB * CPT, CHUNK)
    zeros = jnp.zeros((ROWS_PER_TILE, H), jnp.float32)

    b1r = b1.reshape(1, D)
    b2r = b2.reshape(1, D)
    gw = gate_w.reshape(1, D)
    del gate_b

    ht0, ht1 = _tc_in(x, W1, b1r)
    a0, a1 = _sc_agg(ht0, ht1, src, dst, zeros)
    ht0, ht1 = _tc_mid(a0, a1, W2, b2r)
    a0, a1 = _sc_agg(ht0, ht1, src, dst, zeros)
    return _tc_pool(a0, a1, batch3, gw)


# speedup vs baseline: 1.3925x; 1.0208x over previous
import functools

import jax
import jax.numpy as jnp
from jax import lax
from jax.experimental import pallas as pl
from jax.experimental.pallas import tpu as pltpu
from jax.experimental.pallas import tpu_sc as plsc

NN = 10000
D = 256
H = 128
G = 64
BN = 1000
NB = NN // BN
NSUB = 16
CHUNK = 128
CPT = 80
IB = 16
NBLK = CPT // IB
EP = NSUB * CPT * CHUNK
ROWS_PER_TILE = 632
NSC = NSUB * ROWS_PER_TILE
NACC = NSC

_MAXNORM = 1.0 - 4e-3
_EPS = 1e-15


def _rnorm(x):
    return jnp.maximum(jnp.sqrt(jnp.sum(x * x, axis=-1, keepdims=True)), _EPS)


def _artanh(x):
    x = jnp.clip(x, -1.0 + 1e-7, 1.0 - 1e-7)
    return 0.5 * jnp.log((1.0 + x) / (1.0 - x))


def _proj(x):
    n = _rnorm(x)
    return x * jnp.minimum(1.0, _MAXNORM / n)


def _proj_expmap0(u):
    un = _rnorm(u)
    return u * (jnp.minimum(jnp.tanh(un), _MAXNORM) / un)


def _logmap0(p):
    pn = _rnorm(p)
    return p * (_artanh(pn) / pn)


def _rowsum(v):
    return jnp.sum(v, axis=-1, keepdims=True)


def _norm_rnorm(x):
    s = jnp.maximum(_rowsum(x * x), _EPS * _EPS)
    rin = lax.rsqrt(s)
    return s * rin, rin


def _proj_expmap0_n(u):
    un, rin = _norm_rnorm(u)
    t = jnp.minimum(jnp.tanh(un), _MAXNORM)
    return u * (t * rin), jnp.maximum(t, _EPS)


def _linear_logmap(w, b, h, hn):
    mx = lax.dot_general(h, w, (((1,), (1,)), ((), ())),
                         preferred_element_type=jnp.float32)
    mxn, rmxn = _norm_rnorm(mx)
    al = jnp.minimum(jnp.tanh(mxn / hn * _artanh(hn)), _MAXNORM)
    mv = mx * (al * rmxn)
    x2 = al * al
    hb, _ = _proj_expmap0_n(b)
    y2 = jnp.sum(hb * hb, -1, keepdims=True)
    xy = _rowsum(mv * hb)
    num = (1.0 + 2.0 * xy + y2) * mv + (1.0 - x2) * hb
    rden = 1.0 / jnp.maximum(1.0 + 2.0 * xy + x2 * y2, _EPS)
    nn, rnn = _norm_rnorm(num)
    rn = jnp.maximum(jnp.minimum(nn * rden, _MAXNORM), _EPS)
    return num * (jnp.minimum(rden, _MAXNORM * rnn) * (_artanh(rn) / rn))


def _post_agg(agg):
    n1, rin1 = _norm_rnorm(agg)
    t1 = jnp.minimum(jnp.tanh(n1), _MAXNORM)
    t1c = jnp.maximum(t1, _EPS)
    ht = jax.nn.relu(agg * ((t1 * rin1) * (_artanh(t1c) / t1c)))
    return _proj_expmap0_n(ht)


def _tc_in_body(x_ref, w_ref, b_ref, o0_ref, o1_ref):
    h, hn = _proj_expmap0_n(x_ref[...])
    ht = _linear_logmap(w_ref[...], b_ref[...], h, hn)
    o0_ref[...] = ht[:, :H]
    o1_ref[...] = ht[:, H:]


def _tc_in(xp, w, b):
    return pl.pallas_call(
        _tc_in_body,
        grid=(NB,),
        in_specs=[
            pl.BlockSpec((BN, D), lambda i: (i, 0)),
            pl.BlockSpec((D, D), lambda i: (0, 0)),
            pl.BlockSpec((1, D), lambda i: (0, 0)),
        ],
        out_specs=[
            pl.BlockSpec((BN, H), lambda i: (i, 0)),
            pl.BlockSpec((BN, H), lambda i: (i, 0)),
        ],
        out_shape=[jax.ShapeDtypeStruct((NN, H), jnp.float32)] * 2,
    )(xp, w, b)


def _tc_mid_body(a0_ref, a1_ref, w_ref, b_ref, o0_ref, o1_ref):
    agg = jnp.concatenate([a0_ref[...], a1_ref[...]], axis=1)
    h, hn = _post_agg(agg)
    ht = _linear_logmap(w_ref[...], b_ref[...], h, hn)
    o0_ref[...] = ht[:, :H]
    o1_ref[...] = ht[:, H:]


def _tc_mid(a0, a1, w, b):
    return pl.pallas_call(
        _tc_mid_body,
        grid=(NB,),
        in_specs=[
            pl.BlockSpec((BN, H), lambda i: (i, 0)),
            pl.BlockSpec((BN, H), lambda i: (i, 0)),
            pl.BlockSpec((D, D), lambda i: (0, 0)),
            pl.BlockSpec((1, D), lambda i: (0, 0)),
        ],
        out_specs=[
            pl.BlockSpec((BN, H), lambda i: (i, 0)),
            pl.BlockSpec((BN, H), lambda i: (i, 0)),
        ],
        out_shape=[jax.ShapeDtypeStruct((NN, H), jnp.float32)] * 2,
    )(a0, a1, w, b)


def _tc_pool_body(a0_ref, a1_ref, batch_ref, gw_ref, out_ref,
                  smax_s, den_s, num_s):
    j = pl.program_id(0)

    agg = jnp.concatenate([a0_ref[...], a1_ref[...]], axis=1)
    h, _ = _post_agg(agg)
    gw = gw_ref[...]
    gl = lax.dot_general(gw, h, (((1,), (1,)), ((), ())),
                         preferred_element_type=jnp.float32)
    b2d = batch_ref[...].reshape(1, BN)
    seg = lax.broadcasted_iota(jnp.int32, (G, BN), 0)
    mask = seg == b2d

    @pl.when(j == 0)
    def _():
        smax_s[...] = jnp.full_like(smax_s[...], -1e30)
        den_s[...] = jnp.zeros_like(den_s[...])
        num_s[...] = jnp.zeros_like(num_s[...])

    bm = jnp.max(jnp.where(mask, gl, -1e30), axis=1, keepdims=True)
    m_old = jnp.max(smax_s[...], axis=1, keepdims=True)
    m_new = jnp.maximum(m_old, bm)
    scale = jnp.exp(m_old - m_new)
    e = jnp.where(mask, jnp.exp(gl - m_new), 0.0)
    smax_s[...] = jnp.broadcast_to(m_new, smax_s.shape)
    den_s[...] = den_s[...] * scale + jnp.sum(e, axis=1, keepdims=True)
    num_s[...] = num_s[...] * scale + lax.dot_general(
        e, h, (((1,), (0,)), ((), ())), preferred_element_type=jnp.float32)

    @pl.when(j == NB - 1)
    def _():
        den = jnp.max(den_s[...], axis=1, keepdims=True)
        out_ref[...] = num_s[...] / (den + 1e-16)


def _tc_pool(a0, a1, batch3, gw):
    return pl.pallas_call(
        _tc_pool_body,
        grid=(NB,),
        in_specs=[
            pl.BlockSpec((BN, H), lambda j: (j, 0)),
            pl.BlockSpec((BN, H), lambda j: (j, 0)),
            pl.BlockSpec((1, 1, BN), lambda j: (j, 0, 0)),
            pl.BlockSpec((1, D), lambda j: (0, 0)),
        ],
        out_specs=pl.BlockSpec((G, D), lambda j: (0, 0)),
        out_shape=jax.ShapeDtypeStruct((G, D), jnp.float32),
        scratch_shapes=[
            pltpu.VMEM((G, 128), jnp.float32),
            pltpu.VMEM((G, 128), jnp.float32),
            pltpu.VMEM((G, D), jnp.float32),
        ],
    )(a0, a1, batch3, gw)


def _sc_agg_body(ht0, ht1, src_hbm, dst_hbm, zeros_hbm, o0, o1,
                 src_bufs, dst_bufs, rows0, rows1, acc, sem0, sem1, isem):
    c = lax.axis_index("c")
    s = lax.axis_index("s")

    pltpu.sync_copy(zeros_hbm, acc.at[pl.ds(s * ROWS_PER_TILE, ROWS_PER_TILE)])

    def run(table, out_ref):
        base = s * CPT
        pltpu.sync_copy(src_hbm.at[pl.ds(base, IB)], src_bufs[0])
        pltpu.sync_copy(dst_hbm.at[pl.ds(base, IB)], dst_bufs[0])
        pltpu.async_copy(table.at[src_bufs[0].at[0]], rows0, sem0)
        plsc.subcore_barrier()

        for k in range(NBLK):
            src_v, dst_v = src_bufs[k % 2], dst_bufs[k % 2]
            if k + 1 < NBLK:
                nsrc, ndst = src_bufs[(k + 1) % 2], dst_bufs[(k + 1) % 2]
                pltpu.async_copy(src_hbm.at[pl.ds(base + (k + 1) * IB, IB)],
                                 nsrc, isem)
                pltpu.async_copy(dst_hbm.at[pl.ds(base + (k + 1) * IB, IB)],
                                 ndst, isem)

            @pl.loop(0, IB // 2)
            def _(i):
                j0 = 2 * i
                pltpu.async_copy(table.at[src_v.at[j0 + 1]], rows1, sem1)
                pltpu.make_async_copy(table.at[src_v.at[j0]], rows0, sem0).wait()
                pltpu.sync_copy(rows0, acc.at[dst_v.at[j0]], add=True)

                @pl.when(j0 + 2 < IB)
                def _():
                    pltpu.async_copy(table.at[src_v.at[j0 + 2]], rows0, sem0)

                pltpu.make_async_copy(table.at[src_v.at[j0 + 1]], rows1, sem1).wait()
                pltpu.sync_copy(rows1, acc.at[dst_v.at[j0 + 1]], add=True)

            if k + 1 < NBLK:
                pltpu.make_async_copy(src_hbm.at[pl.ds(0, IB)], nsrc, isem).wait()
                pltpu.make_async_copy(dst_hbm.at[pl.ds(0, IB)], ndst, isem).wait()
                pltpu.async_copy(table.at[nsrc.at[0]], rows0, sem0)

        plsc.subcore_barrier()
        wb = s * ROWS_PER_TILE
        pltpu.sync_copy(acc.at[pl.ds(wb, ROWS_PER_TILE)],
                        out_ref.at[pl.ds(wb, ROWS_PER_TILE)])

    @pl.when(c == 0)
    def _():
        run(ht0, o0)

    @pl.when(c == 1)
    def _():
        run(ht1, o1)


@functools.cache
def _make_sc_agg():
    return pl.kernel(
        _sc_agg_body,
        out_type=[jax.ShapeDtypeStruct((NSC, H), jnp.float32)] * 2,
        mesh=plsc.VectorSubcoreMesh(core_axis_name="c", subcore_axis_name="s"),
        scratch_types=[
            [pltpu.VMEM((IB, CHUNK), jnp.int32) for _ in range(2)],
            [pltpu.VMEM((IB, CHUNK), jnp.int32) for _ in range(2)],
            pltpu.VMEM((CHUNK, H), jnp.float32),
            pltpu.VMEM((CHUNK, H), jnp.float32),
            pltpu.VMEM_SHARED((NACC, H), jnp.float32),
            pltpu.SemaphoreType.DMA,
            pltpu.SemaphoreType.DMA,
            pltpu.SemaphoreType.DMA,
        ],
    )


def _sc_agg(ht0, ht1, src2d, dst2d, zeros):
    return _make_sc_agg()(ht0, ht1, src2d, dst2d, zeros)


def kernel(x, edge_index, batch, W1, b1, W2, b2, gate_w, gate_b):
    n = x.shape[0]
    e = edge_index.shape[1]

    batch3 = batch.astype(jnp.int32).reshape(NB, 1, BN)

    pad = EP - e
    filler = jnp.arange(pad, dtype=jnp.int32)
    src = jnp.concatenate([edge_index[0], filler % n]).reshape(NSUB * CPT, CHUNK)
    dst = jnp.concatenate([edge_index[1], n + (filler % 8)]).reshape(NSUB * CPT, CHUNK)
    zeros = jnp.zeros((ROWS_PER_TILE, H), jnp.float32)

    b1r = b1.reshape(1, D)
    b2r = b2.reshape(1, D)
    gw = gate_w.reshape(1, D)
    del gate_b

    ht0, ht1 = _tc_in(x, W1, b1r)
    a0, a1 = _sc_agg(ht0, ht1, src, dst, zeros)
    ht0, ht1 = _tc_mid(a0, a1, W2, b2r)
    a0, a1 = _sc_agg(ht0, ht1, src, dst, zeros)
    return _tc_pool(a0, a1, batch3, gw)


# speedup vs baseline: 1.3989x; 1.0046x over previous
import functools

import jax
import jax.numpy as jnp
from jax import lax
from jax.experimental import pallas as pl
from jax.experimental.pallas import tpu as pltpu
from jax.experimental.pallas import tpu_sc as plsc

NN = 10000
D = 256
H = 128
G = 64
BN = 2000
NB = NN // BN
NSUB = 16
CHUNK = 128
CPT = 80
IB = 16
NBLK = CPT // IB
EP = NSUB * CPT * CHUNK
ROWS_PER_TILE = 632
NSC = NSUB * ROWS_PER_TILE
NACC = NSC

_MAXNORM = 1.0 - 4e-3
_EPS = 1e-15


def _rnorm(x):
    return jnp.maximum(jnp.sqrt(jnp.sum(x * x, axis=-1, keepdims=True)), _EPS)


def _artanh(x):
    x = jnp.clip(x, -1.0 + 1e-7, 1.0 - 1e-7)
    return 0.5 * jnp.log((1.0 + x) / (1.0 - x))


def _proj(x):
    n = _rnorm(x)
    return x * jnp.minimum(1.0, _MAXNORM / n)


def _proj_expmap0(u):
    un = _rnorm(u)
    return u * (jnp.minimum(jnp.tanh(un), _MAXNORM) / un)


def _logmap0(p):
    pn = _rnorm(p)
    return p * (_artanh(pn) / pn)


def _rowsum(v):
    return jnp.sum(v, axis=-1, keepdims=True)


def _norm_rnorm(x):
    s = jnp.maximum(_rowsum(x * x), _EPS * _EPS)
    rin = lax.rsqrt(s)
    return s * rin, rin


def _proj_expmap0_n(u):
    un, rin = _norm_rnorm(u)
    t = jnp.minimum(jnp.tanh(un), _MAXNORM)
    return u * (t * rin), jnp.maximum(t, _EPS)


def _linear_logmap(w, b, h, hn):
    mx = lax.dot_general(h, w, (((1,), (1,)), ((), ())),
                         preferred_element_type=jnp.float32)
    mxn, rmxn = _norm_rnorm(mx)
    al = jnp.minimum(jnp.tanh(mxn / hn * _artanh(hn)), _MAXNORM)
    mv = mx * (al * rmxn)
    x2 = al * al
    hb, _ = _proj_expmap0_n(b)
    y2 = jnp.sum(hb * hb, -1, keepdims=True)
    xy = _rowsum(mv * hb)
    num = (1.0 + 2.0 * xy + y2) * mv + (1.0 - x2) * hb
    rden = 1.0 / jnp.maximum(1.0 + 2.0 * xy + x2 * y2, _EPS)
    nn, rnn = _norm_rnorm(num)
    rn = jnp.maximum(jnp.minimum(nn * rden, _MAXNORM), _EPS)
    return num * (jnp.minimum(rden, _MAXNORM * rnn) * (_artanh(rn) / rn))


def _post_agg(agg):
    n1, rin1 = _norm_rnorm(agg)
    t1 = jnp.minimum(jnp.tanh(n1), _MAXNORM)
    t1c = jnp.maximum(t1, _EPS)
    ht = jax.nn.relu(agg * ((t1 * rin1) * (_artanh(t1c) / t1c)))
    return _proj_expmap0_n(ht)


def _tc_in_body(x_ref, w_ref, b_ref, o0_ref, o1_ref):
    h, hn = _proj_expmap0_n(x_ref[...])
    ht = _linear_logmap(w_ref[...], b_ref[...], h, hn)
    o0_ref[...] = ht[:, :H]
    o1_ref[...] = ht[:, H:]


def _tc_in(xp, w, b):
    return pl.pallas_call(
        _tc_in_body,
        grid=(NB,),
        in_specs=[
            pl.BlockSpec((BN, D), lambda i: (i, 0)),
            pl.BlockSpec((D, D), lambda i: (0, 0)),
            pl.BlockSpec((1, D), lambda i: (0, 0)),
        ],
        out_specs=[
            pl.BlockSpec((BN, H), lambda i: (i, 0)),
            pl.BlockSpec((BN, H), lambda i: (i, 0)),
        ],
        out_shape=[jax.ShapeDtypeStruct((NN, H), jnp.float32)] * 2,
    )(xp, w, b)


def _tc_mid_body(a0_ref, a1_ref, w_ref, b_ref, o0_ref, o1_ref):
    agg = jnp.concatenate([a0_ref[...], a1_ref[...]], axis=1)
    h, hn = _post_agg(agg)
    ht = _linear_logmap(w_ref[...], b_ref[...], h, hn)
    o0_ref[...] = ht[:, :H]
    o1_ref[...] = ht[:, H:]


def _tc_mid(a0, a1, w, b):
    return pl.pallas_call(
        _tc_mid_body,
        grid=(NB,),
        in_specs=[
            pl.BlockSpec((BN, H), lambda i: (i, 0)),
            pl.BlockSpec((BN, H), lambda i: (i, 0)),
            pl.BlockSpec((D, D), lambda i: (0, 0)),
            pl.BlockSpec((1, D), lambda i: (0, 0)),
        ],
        out_specs=[
            pl.BlockSpec((BN, H), lambda i: (i, 0)),
            pl.BlockSpec((BN, H), lambda i: (i, 0)),
        ],
        out_shape=[jax.ShapeDtypeStruct((NN, H), jnp.float32)] * 2,
    )(a0, a1, w, b)


def _tc_pool_body(a0_ref, a1_ref, batch_ref, gw_ref, out_ref,
                  smax_s, den_s, num_s):
    j = pl.program_id(0)

    agg = jnp.concatenate([a0_ref[...], a1_ref[...]], axis=1)
    h, _ = _post_agg(agg)
    gw = gw_ref[...]
    gl = lax.dot_general(gw, h, (((1,), (1,)), ((), ())),
                         preferred_element_type=jnp.float32)
    b2d = batch_ref[...].reshape(1, BN)
    seg = lax.broadcasted_iota(jnp.int32, (G, BN), 0)
    mask = seg == b2d

    @pl.when(j == 0)
    def _():
        smax_s[...] = jnp.full_like(smax_s[...], -1e30)
        den_s[...] = jnp.zeros_like(den_s[...])
        num_s[...] = jnp.zeros_like(num_s[...])

    bm = jnp.max(jnp.where(mask, gl, -1e30), axis=1, keepdims=True)
    m_old = jnp.max(smax_s[...], axis=1, keepdims=True)
    m_new = jnp.maximum(m_old, bm)
    scale = jnp.exp(m_old - m_new)
    e = jnp.where(mask, jnp.exp(gl - m_new), 0.0)
    smax_s[...] = jnp.broadcast_to(m_new, smax_s.shape)
    den_s[...] = den_s[...] * scale + jnp.sum(e, axis=1, keepdims=True)
    num_s[...] = num_s[...] * scale + lax.dot_general(
        e, h, (((1,), (0,)), ((), ())), preferred_element_type=jnp.float32)

    @pl.when(j == NB - 1)
    def _():
        den = jnp.max(den_s[...], axis=1, keepdims=True)
        out_ref[...] = num_s[...] / (den + 1e-16)


def _tc_pool(a0, a1, batch3, gw):
    return pl.pallas_call(
        _tc_pool_body,
        grid=(NB,),
        in_specs=[
            pl.BlockSpec((BN, H), lambda j: (j, 0)),
            pl.BlockSpec((BN, H), lambda j: (j, 0)),
            pl.BlockSpec((1, 1, BN), lambda j: (j, 0, 0)),
            pl.BlockSpec((1, D), lambda j: (0, 0)),
        ],
        out_specs=pl.BlockSpec((G, D), lambda j: (0, 0)),
        out_shape=jax.ShapeDtypeStruct((G, D), jnp.float32),
        scratch_shapes=[
            pltpu.VMEM((G, 128), jnp.float32),
            pltpu.VMEM((G, 128), jnp.float32),
            pltpu.VMEM((G, D), jnp.float32),
        ],
    )(a0, a1, batch3, gw)


def _sc_agg_body(ht0, ht1, src_hbm, dst_hbm, zeros_hbm, o0, o1,
                 src_bufs, dst_bufs, rows0, rows1, acc, sem0, sem1, isem):
    c = lax.axis_index("c")
    s = lax.axis_index("s")

    pltpu.sync_copy(zeros_hbm, acc.at[pl.ds(s * ROWS_PER_TILE, ROWS_PER_TILE)])

    def run(table, out_ref):
        base = s * CPT
        pltpu.sync_copy(src_hbm.at[pl.ds(base, IB)], src_bufs[0])
        pltpu.sync_copy(dst_hbm.at[pl.ds(base, IB)], dst_bufs[0])
        pltpu.async_copy(table.at[src_bufs[0].at[0]], rows0, sem0)
        plsc.subcore_barrier()

        for k in range(NBLK):
            src_v, dst_v = src_bufs[k % 2], dst_bufs[k % 2]
            if k + 1 < NBLK:
                nsrc, ndst = src_bufs[(k + 1) % 2], dst_bufs[(k + 1) % 2]
                pltpu.async_copy(src_hbm.at[pl.ds(base + (k + 1) * IB, IB)],
                                 nsrc, isem)
                pltpu.async_copy(dst_hbm.at[pl.ds(base + (k + 1) * IB, IB)],
                                 ndst, isem)

            @pl.loop(0, IB // 2)
            def _(i):
                j0 = 2 * i
                pltpu.async_copy(table.at[src_v.at[j0 + 1]], rows1, sem1)
                pltpu.make_async_copy(table.at[src_v.at[j0]], rows0, sem0).wait()
                pltpu.sync_copy(rows0, acc.at[dst_v.at[j0]], add=True)

                @pl.when(j0 + 2 < IB)
                def _():
                    pltpu.async_copy(table.at[src_v.at[j0 + 2]], rows0, sem0)

                pltpu.make_async_copy(table.at[src_v.at[j0 + 1]], rows1, sem1).wait()
                pltpu.sync_copy(rows1, acc.at[dst_v.at[j0 + 1]], add=True)

            if k + 1 < NBLK:
                pltpu.make_async_copy(src_hbm.at[pl.ds(0, IB)], nsrc, isem).wait()
                pltpu.make_async_copy(dst_hbm.at[pl.ds(0, IB)], ndst, isem).wait()
                pltpu.async_copy(table.at[nsrc.at[0]], rows0, sem0)

        plsc.subcore_barrier()
        wb = s * ROWS_PER_TILE
        pltpu.sync_copy(acc.at[pl.ds(wb, ROWS_PER_TILE)],
                        out_ref.at[pl.ds(wb, ROWS_PER_TILE)])

    @pl.when(c == 0)
    def _():
        run(ht0, o0)

    @pl.when(c == 1)
    def _():
        run(ht1, o1)


@functools.cache
def _make_sc_agg():
    return pl.kernel(
        _sc_agg_body,
        out_type=[jax.ShapeDtypeStruct((NSC, H), jnp.float32)] * 2,
        mesh=plsc.VectorSubcoreMesh(core_axis_name="c", subcore_axis_name="s"),
        scratch_types=[
            [pltpu.VMEM((IB, CHUNK), jnp.int32) for _ in range(2)],
            [pltpu.VMEM((IB, CHUNK), jnp.int32) for _ in range(2)],
            pltpu.VMEM((CHUNK, H), jnp.float32),
            pltpu.VMEM((CHUNK, H), jnp.float32),
            pltpu.VMEM_SHARED((NACC, H), jnp.float32),
            pltpu.SemaphoreType.DMA,
            pltpu.SemaphoreType.DMA,
            pltpu.SemaphoreType.DMA,
        ],
    )


def _sc_agg(ht0, ht1, src2d, dst2d, zeros):
    return _make_sc_agg()(ht0, ht1, src2d, dst2d, zeros)


def kernel(x, edge_index, batch, W1, b1, W2, b2, gate_w, gate_b):
    n = x.shape[0]
    e = edge_index.shape[1]

    batch3 = batch.astype(jnp.int32).reshape(NB, 1, BN)

    pad = EP - e
    filler = jnp.arange(pad, dtype=jnp.int32)
    src = jnp.concatenate([edge_index[0], filler % n]).reshape(NSUB * CPT, CHUNK)
    dst = jnp.concatenate([edge_index[1], n + (filler % 8)]).reshape(NSUB * CPT, CHUNK)
    zeros = jnp.zeros((ROWS_PER_TILE, H), jnp.float32)

    b1r = b1.reshape(1, D)
    b2r = b2.reshape(1, D)
    gw = gate_w.reshape(1, D)
    del gate_b

    ht0, ht1 = _tc_in(x, W1, b1r)
    a0, a1 = _sc_agg(ht0, ht1, src, dst, zeros)
    ht0, ht1 = _tc_mid(a0, a1, W2, b2r)
    a0, a1 = _sc_agg(ht0, ht1, src, dst, zeros)
    return _tc_pool(a0, a1, batch3, gw)


# speedup vs baseline: 1.4009x; 1.0015x over previous
import functools

import jax
import jax.numpy as jnp
from jax import lax
from jax.experimental import pallas as pl
from jax.experimental.pallas import tpu as pltpu
from jax.experimental.pallas import tpu_sc as plsc

NN = 10000
D = 256
H = 128
G = 64
BN = 2000
NB = NN // BN
NSUB = 16
CHUNK = 128
CPT = 80
IB = 16
NBLK = CPT // IB
EP = NSUB * CPT * CHUNK
ROWS_PER_TILE = 632
NSC = NSUB * ROWS_PER_TILE
NACC = NSC

_MAXNORM = 1.0 - 4e-3
_EPS = 1e-15


def _artanh(x):
    x = jnp.clip(x, -1.0 + 1e-7, 1.0 - 1e-7)
    return 0.5 * jnp.log((1.0 + x) / (1.0 - x))


def _rowsum(v):
    return jnp.sum(v, axis=-1, keepdims=True)


def _norm_rnorm(x):
    s = jnp.maximum(_rowsum(x * x), _EPS * _EPS)
    rin = lax.rsqrt(s)
    return s * rin, rin


def _proj_expmap0_n(u):
    un, rin = _norm_rnorm(u)
    t = jnp.minimum(jnp.tanh(un), _MAXNORM)
    return u * (t * rin), jnp.maximum(t, _EPS)


def _linear_logmap(w, b, h, hn):
    mx = lax.dot_general(h, w, (((1,), (1,)), ((), ())),
                         preferred_element_type=jnp.float32)
    mxn, rmxn = _norm_rnorm(mx)
    al = jnp.minimum(jnp.tanh(mxn / hn * _artanh(hn)), _MAXNORM)
    mv = mx * (al * rmxn)
    x2 = al * al
    hb, _ = _proj_expmap0_n(b)
    y2 = jnp.sum(hb * hb, -1, keepdims=True)
    xy = _rowsum(mv * hb)
    num = (1.0 + 2.0 * xy + y2) * mv + (1.0 - x2) * hb
    rden = 1.0 / jnp.maximum(1.0 + 2.0 * xy + x2 * y2, _EPS)
    nn, rnn = _norm_rnorm(num)
    rn = jnp.maximum(jnp.minimum(nn * rden, _MAXNORM), _EPS)
    return num * (jnp.minimum(rden, _MAXNORM * rnn) * (_artanh(rn) / rn))


def _post_agg(agg):
    n1, rin1 = _norm_rnorm(agg)
    t1 = jnp.minimum(jnp.tanh(n1), _MAXNORM)
    t1c = jnp.maximum(t1, _EPS)
    ht = jax.nn.relu(agg * ((t1 * rin1) * (_artanh(t1c) / t1c)))
    return _proj_expmap0_n(ht)


def _tc_in_body(x_ref, w_ref, b_ref, o0_ref, o1_ref):
    h, hn = _proj_expmap0_n(x_ref[...])
    ht = _linear_logmap(w_ref[...], b_ref[...], h, hn)
    o0_ref[...] = ht[:, :H]
    o1_ref[...] = ht[:, H:]


def _tc_in(xp, w, b):
    return pl.pallas_call(
        _tc_in_body,
        grid=(NB,),
        in_specs=[
            pl.BlockSpec((BN, D), lambda i: (i, 0)),
            pl.BlockSpec((D, D), lambda i: (0, 0)),
            pl.BlockSpec((1, D), lambda i: (0, 0)),
        ],
        out_specs=[
            pl.BlockSpec((BN, H), lambda i: (i, 0)),
            pl.BlockSpec((BN, H), lambda i: (i, 0)),
        ],
        out_shape=[jax.ShapeDtypeStruct((NN, H), jnp.float32)] * 2,
    )(xp, w, b)


def _tc_mid_body(a0_ref, a1_ref, w_ref, b_ref, o0_ref, o1_ref):
    agg = jnp.concatenate([a0_ref[...], a1_ref[...]], axis=1)
    h, hn = _post_agg(agg)
    ht = _linear_logmap(w_ref[...], b_ref[...], h, hn)
    o0_ref[...] = ht[:, :H]
    o1_ref[...] = ht[:, H:]


def _tc_mid(a0, a1, w, b):
    return pl.pallas_call(
        _tc_mid_body,
        grid=(NB,),
        in_specs=[
            pl.BlockSpec((BN, H), lambda i: (i, 0)),
            pl.BlockSpec((BN, H), lambda i: (i, 0)),
            pl.BlockSpec((D, D), lambda i: (0, 0)),
            pl.BlockSpec((1, D), lambda i: (0, 0)),
        ],
        out_specs=[
            pl.BlockSpec((BN, H), lambda i: (i, 0)),
            pl.BlockSpec((BN, H), lambda i: (i, 0)),
        ],
        out_shape=[jax.ShapeDtypeStruct((NN, H), jnp.float32)] * 2,
    )(a0, a1, w, b)


def _tc_pool_body(a0_ref, a1_ref, batch_ref, gw_ref, out_ref,
                  smax_s, den_s, num_s):
    j = pl.program_id(0)

    agg = jnp.concatenate([a0_ref[...], a1_ref[...]], axis=1)
    h, _ = _post_agg(agg)
    gw = gw_ref[...]
    gl = lax.dot_general(gw, h, (((1,), (1,)), ((), ())),
                         preferred_element_type=jnp.float32)
    b2d = batch_ref[...].reshape(1, BN)
    seg = lax.broadcasted_iota(jnp.int32, (G, BN), 0)
    mask = seg == b2d

    @pl.when(j == 0)
    def _():
        smax_s[...] = jnp.full_like(smax_s[...], -1e30)
        den_s[...] = jnp.zeros_like(den_s[...])
        num_s[...] = jnp.zeros_like(num_s[...])

    bm = jnp.max(jnp.where(mask, gl, -1e30), axis=1, keepdims=True)
    m_old = jnp.max(smax_s[...], axis=1, keepdims=True)
    m_new = jnp.maximum(m_old, bm)
    scale = jnp.exp(m_old - m_new)
    e = jnp.where(mask, jnp.exp(gl - m_new), 0.0)
    smax_s[...] = jnp.broadcast_to(m_new, smax_s.shape)
    den_s[...] = den_s[...] * scale + jnp.sum(e, axis=1, keepdims=True)
    num_s[...] = num_s[...] * scale + lax.dot_general(
        e, h, (((1,), (0,)), ((), ())), preferred_element_type=jnp.float32)

    @pl.when(j == NB - 1)
    def _():
        den = jnp.max(den_s[...], axis=1, keepdims=True)
        out_ref[...] = num_s[...] / (den + 1e-16)


def _tc_pool(a0, a1, batch3, gw):
    return pl.pallas_call(
        _tc_pool_body,
        grid=(NB,),
        in_specs=[
            pl.BlockSpec((BN, H), lambda j: (j, 0)),
            pl.BlockSpec((BN, H), lambda j: (j, 0)),
            pl.BlockSpec((1, 1, BN), lambda j: (j, 0, 0)),
            pl.BlockSpec((1, D), lambda j: (0, 0)),
        ],
        out_specs=pl.BlockSpec((G, D), lambda j: (0, 0)),
        out_shape=jax.ShapeDtypeStruct((G, D), jnp.float32),
        scratch_shapes=[
            pltpu.VMEM((G, 128), jnp.float32),
            pltpu.VMEM((G, 128), jnp.float32),
            pltpu.VMEM((G, D), jnp.float32),
        ],
    )(a0, a1, batch3, gw)


def _sc_agg_body(ht0, ht1, src_hbm, dst_hbm, zeros_hbm, o0, o1,
                 src_bufs, dst_bufs, rows0, rows1, acc, sem0, sem1, isem):
    c = lax.axis_index("c")
    s = lax.axis_index("s")

    pltpu.sync_copy(zeros_hbm, acc.at[pl.ds(s * ROWS_PER_TILE, ROWS_PER_TILE)])

    def run(table, out_ref):
        base = s * CPT
        pltpu.sync_copy(src_hbm.at[pl.ds(base, IB)], src_bufs[0])
        pltpu.sync_copy(dst_hbm.at[pl.ds(base, IB)], dst_bufs[0])
        pltpu.async_copy(table.at[src_bufs[0].at[0]], rows0, sem0)
        plsc.subcore_barrier()

        for k in range(NBLK):
            src_v, dst_v = src_bufs[k % 2], dst_bufs[k % 2]
            if k + 1 < NBLK:
                nsrc, ndst = src_bufs[(k + 1) % 2], dst_bufs[(k + 1) % 2]
                pltpu.async_copy(src_hbm.at[pl.ds(base + (k + 1) * IB, IB)],
                                 nsrc, isem)
                pltpu.async_copy(dst_hbm.at[pl.ds(base + (k + 1) * IB, IB)],
                                 ndst, isem)

            @pl.loop(0, IB // 2)
            def _(i):
                j0 = 2 * i
                pltpu.async_copy(table.at[src_v.at[j0 + 1]], rows1, sem1)
                pltpu.make_async_copy(table.at[src_v.at[j0]], rows0, sem0).wait()
                pltpu.sync_copy(rows0, acc.at[dst_v.at[j0]], add=True)

                @pl.when(j0 + 2 < IB)
                def _():
                    pltpu.async_copy(table.at[src_v.at[j0 + 2]], rows0, sem0)

                pltpu.make_async_copy(table.at[src_v.at[j0 + 1]], rows1, sem1).wait()
                pltpu.sync_copy(rows1, acc.at[dst_v.at[j0 + 1]], add=True)

            if k + 1 < NBLK:
                pltpu.make_async_copy(src_hbm.at[pl.ds(0, IB)], nsrc, isem).wait()
                pltpu.make_async_copy(dst_hbm.at[pl.ds(0, IB)], ndst, isem).wait()
                pltpu.async_copy(table.at[nsrc.at[0]], rows0, sem0)

        plsc.subcore_barrier()
        wb = s * ROWS_PER_TILE
        pltpu.sync_copy(acc.at[pl.ds(wb, ROWS_PER_TILE)],
                        out_ref.at[pl.ds(wb, ROWS_PER_TILE)])

    @pl.when(c == 0)
    def _():
        run(ht0, o0)

    @pl.when(c == 1)
    def _():
        run(ht1, o1)


@functools.cache
def _make_sc_agg():
    return pl.kernel(
        _sc_agg_body,
        out_type=[jax.ShapeDtypeStruct((NSC, H), jnp.float32)] * 2,
        mesh=plsc.VectorSubcoreMesh(core_axis_name="c", subcore_axis_name="s"),
        scratch_types=[
            [pltpu.VMEM((IB, CHUNK), jnp.int32) for _ in range(2)],
            [pltpu.VMEM((IB, CHUNK), jnp.int32) for _ in range(2)],
            pltpu.VMEM((CHUNK, H), jnp.float32),
            pltpu.VMEM((CHUNK, H), jnp.float32),
            pltpu.VMEM_SHARED((NACC, H), jnp.float32),
            pltpu.SemaphoreType.DMA,
            pltpu.SemaphoreType.DMA,
            pltpu.SemaphoreType.DMA,
        ],
    )


def _sc_agg(ht0, ht1, src2d, dst2d, zeros):
    return _make_sc_agg()(ht0, ht1, src2d, dst2d, zeros)


def kernel(x, edge_index, batch, W1, b1, W2, b2, gate_w, gate_b):
    n = x.shape[0]
    e = edge_index.shape[1]

    batch3 = batch.astype(jnp.int32).reshape(NB, 1, BN)

    pad = EP - e
    filler = jnp.arange(pad, dtype=jnp.int32)
    src = jnp.concatenate([edge_index[0], filler % n]).reshape(NSUB * CPT, CHUNK)
    dst = jnp.concatenate([edge_index[1], n + (filler % 8)]).reshape(NSUB * CPT, CHUNK)
    zeros = jnp.zeros((ROWS_PER_TILE, H), jnp.float32)

    b1r = b1.reshape(1, D)
    b2r = b2.reshape(1, D)
    gw = gate_w.reshape(1, D)
    del gate_b

    ht0, ht1 = _tc_in(x, W1, b1r)
    a0, a1 = _sc_agg(ht0, ht1, src, dst, zeros)
    ht0, ht1 = _tc_mid(a0, a1, W2, b2r)
    a0, a1 = _sc_agg(ht0, ht1, src, dst, zeros)
    return _tc_pool(a0, a1, batch3, gw)
